# Initial kernel scaffold; baseline (speedup 1.0000x reference)
import jax
import jax.numpy as jnp
from jax.experimental import pallas as pl


def kernel(x, cell_id, nodal_values, shape_functions, flag_training, connectivity):
    raise NotImplementedError("write your pallas kernel here")


# speedup vs baseline: 7.2321x; 7.2321x over previous
import functools

import jax
import jax.numpy as jnp
from jax import lax
from jax.experimental import pallas as pl
from jax.experimental.pallas import tpu as pltpu
from jax.experimental.pallas import tpu_sc as plsc

_N_CELLS = 128
_N_NODES = 130
_N_PTS = 16384
_L = 16
_NC = 2
_NS = 16
_NW = _NC * _NS
_PW = _N_PTS // _NW
_STEPS = _PW // _L


@functools.partial(
    pl.kernel,
    out_type=jax.ShapeDtypeStruct((2, _N_PTS), jnp.float32),
    mesh=plsc.VectorSubcoreMesh(core_axis_name="c", subcore_axis_name="s"),
    compiler_params=pltpu.CompilerParams(needs_layout_passes=False),
    scratch_types=[
        pltpu.VMEM((_PW,), jnp.int32),
        pltpu.VMEM((_PW * 3,), jnp.float32),
        pltpu.VMEM((2 * _N_NODES,), jnp.float32),
        pltpu.VMEM((_N_CELLS * 3,), jnp.int32),
        pltpu.VMEM((2 * _PW,), jnp.float32),
    ],
)
def _interp_sc(cid_hbm, sf_hbm, vals_hbm, conn_hbm, out_hbm,
               cid_v, sf_v, vals_v, conn_v, out_v):
    wid = lax.axis_index("s") * _NC + lax.axis_index("c")
    base = wid * _PW
    pltpu.sync_copy(cid_hbm.at[pl.ds(base, _PW)], cid_v)
    pltpu.sync_copy(sf_hbm.at[pl.ds(base * 3, _PW * 3)], sf_v)
    pltpu.sync_copy(vals_hbm, vals_v)
    pltpu.sync_copy(conn_hbm, conn_v)

    lane = lax.iota(jnp.int32, _L)

    def step(i, carry):
        off = i * _L
        cid3 = cid_v[pl.ds(off, _L)] * 3
        sfo = (off + lane) * 3
        nodes = [plsc.load_gather(conn_v, [cid3 + k]) - 1 for k in range(3)]
        ws = [plsc.load_gather(sf_v, [sfo + k]) for k in range(3)]
        for c in range(2):
            cbase = c * _N_NODES
            acc = ws[0] * plsc.load_gather(vals_v, [nodes[0] + cbase])
            acc = acc + ws[1] * plsc.load_gather(vals_v, [nodes[1] + cbase])
            acc = acc + ws[2] * plsc.load_gather(vals_v, [nodes[2] + cbase])
            out_v[pl.ds(c * _PW + off, _L)] = acc
        return carry

    lax.fori_loop(0, _STEPS, step, 0)

    pltpu.sync_copy(out_v.at[pl.ds(0, _PW)], out_hbm.at[0, pl.ds(base, _PW)])
    pltpu.sync_copy(out_v.at[pl.ds(_PW, _PW)], out_hbm.at[1, pl.ds(base, _PW)])


def kernel(x, cell_id, nodal_values, shape_functions, flag_training, connectivity):
    cid = cell_id.astype(jnp.int32)
    sf = shape_functions.reshape(-1)
    vals = nodal_values.reshape(-1)
    conn = connectivity.astype(jnp.int32).reshape(-1)
    return _interp_sc(cid, sf, vals, conn)


# speedup vs baseline: 7.6420x; 1.0567x over previous
import functools

import jax
import jax.numpy as jnp
from jax import lax
from jax.experimental import pallas as pl
from jax.experimental.pallas import tpu as pltpu
from jax.experimental.pallas import tpu_sc as plsc

_N_CELLS = 128
_N_NODES = 130
_N_PTS = 16384
_L = 16
_NC = 2
_NS = 16
_NW = _NC * _NS
_PW = _N_PTS // _NW
_STEPS = _PW // _L


@functools.partial(
    pl.kernel,
    out_type=jax.ShapeDtypeStruct((2, _N_PTS), jnp.float32),
    mesh=plsc.VectorSubcoreMesh(core_axis_name="c", subcore_axis_name="s"),
    compiler_params=pltpu.CompilerParams(needs_layout_passes=False),
    scratch_types=[
        pltpu.VMEM((_PW,), jnp.int32),
        pltpu.VMEM((_PW * 3,), jnp.float32),
        pltpu.VMEM((2 * _N_NODES,), jnp.float32),
        pltpu.VMEM((_N_CELLS * 3,), jnp.int32),
        pltpu.VMEM((2 * _PW,), jnp.float32),
        pltpu.SemaphoreType.DMA,
    ],
)
def _interp_sc(cid_hbm, sf_hbm, vals_hbm, conn_hbm, out_hbm,
               cid_v, sf_v, vals_v, conn_v, out_v, sem):
    wid = lax.axis_index("s") * _NC + lax.axis_index("c")
    base = wid * _PW
    d1 = pltpu.async_copy(cid_hbm.at[pl.ds(base, _PW)], cid_v, sem)
    d2 = pltpu.async_copy(sf_hbm.at[pl.ds(base * 3, _PW * 3)], sf_v, sem)
    d3 = pltpu.async_copy(vals_hbm, vals_v, sem)
    d4 = pltpu.async_copy(conn_hbm, conn_v, sem)
    d1.wait()
    d2.wait()
    d3.wait()
    d4.wait()

    lane = lax.iota(jnp.int32, _L)

    @plsc.parallel_loop(0, _STEPS, 1, unroll=4)
    def step(i):
        off = i * _L
        cid3 = cid_v[pl.ds(off, _L)] * 3
        sfo = (off + lane) * 3
        nodes = [plsc.load_gather(conn_v, [cid3 + k]) - 1 for k in range(3)]
        ws = [plsc.load_gather(sf_v, [sfo + k]) for k in range(3)]
        for c in range(2):
            cbase = c * _N_NODES
            acc = ws[0] * plsc.load_gather(vals_v, [nodes[0] + cbase])
            acc = acc + ws[1] * plsc.load_gather(vals_v, [nodes[1] + cbase])
            acc = acc + ws[2] * plsc.load_gather(vals_v, [nodes[2] + cbase])
            out_v[pl.ds(c * _PW + off, _L)] = acc

    o1 = pltpu.async_copy(out_v.at[pl.ds(0, _PW)], out_hbm.at[0, pl.ds(base, _PW)], sem)
    o2 = pltpu.async_copy(out_v.at[pl.ds(_PW, _PW)], out_hbm.at[1, pl.ds(base, _PW)], sem)
    o1.wait()
    o2.wait()


def kernel(x, cell_id, nodal_values, shape_functions, flag_training, connectivity):
    cid = cell_id.astype(jnp.int32)
    sf = shape_functions.reshape(-1)
    vals = nodal_values.reshape(-1)
    conn = connectivity.astype(jnp.int32).reshape(-1)
    return _interp_sc(cid, sf, vals, conn)


# speedup vs baseline: 8.4352x; 1.1038x over previous
import functools

import jax
import jax.numpy as jnp
from jax import lax
from jax.experimental import pallas as pl
from jax.experimental.pallas import tpu as pltpu
from jax.experimental.pallas import tpu_sc as plsc

_N_CELLS = 128
_N_NODES = 130
_N_PTS = 16384
_L = 16
_NC = 2
_NS = 16
_NW = _NC * _NS
_PW = _N_PTS // _NW
_STEPS = _PW // _L


@functools.partial(
    pl.kernel,
    out_type=jax.ShapeDtypeStruct((2, _N_PTS), jnp.float32),
    mesh=plsc.VectorSubcoreMesh(core_axis_name="c", subcore_axis_name="s"),
    compiler_params=pltpu.CompilerParams(needs_layout_passes=False),
    scratch_types=[
        pltpu.VMEM((_PW,), jnp.int32),
        pltpu.VMEM((_PW, 3), jnp.float32),
        pltpu.VMEM((2 * _N_NODES,), jnp.float32),
        pltpu.VMEM((3 * _N_CELLS,), jnp.int32),
        pltpu.VMEM((2 * _PW,), jnp.float32),
        pltpu.SemaphoreType.DMA,
    ],
)
def _interp_sc(cid_hbm, sf_hbm, vals_hbm, conn_hbm, out_hbm,
               cid_v, sf_v, vals_v, conn_v, out_v, sem):
    wid = lax.axis_index("s") * _NC + lax.axis_index("c")
    base = wid * _PW
    copies = [
        pltpu.async_copy(cid_hbm.at[pl.ds(base, _PW)], cid_v, sem),
        pltpu.async_copy(sf_hbm.at[pl.ds(base, _PW), :], sf_v, sem),
        pltpu.async_copy(vals_hbm, vals_v, sem),
        pltpu.async_copy(conn_hbm, conn_v, sem),
    ]
    for cp in copies:
        cp.wait()

    lane = lax.iota(jnp.int32, _L)

    @plsc.parallel_loop(0, _STEPS, 1, unroll=4)
    def step(i):
        off = i * _L
        cid3 = cid_v[pl.ds(off, _L)] * 3
        rows = off + lane
        nodes = [plsc.load_gather(conn_v, [cid3 + k]) - 1 for k in range(3)]
        ws = [plsc.load_gather(sf_v, [rows, jnp.full((_L,), k, jnp.int32)])
              for k in range(3)]
        for c in range(2):
            cbase = c * _N_NODES
            acc = ws[0] * plsc.load_gather(vals_v, [nodes[0] + cbase])
            acc = acc + ws[1] * plsc.load_gather(vals_v, [nodes[1] + cbase])
            acc = acc + ws[2] * plsc.load_gather(vals_v, [nodes[2] + cbase])
            out_v[pl.ds(c * _PW + off, _L)] = acc

    o1 = pltpu.async_copy(out_v.at[pl.ds(0, _PW)], out_hbm.at[0, pl.ds(base, _PW)], sem)
    o2 = pltpu.async_copy(out_v.at[pl.ds(_PW, _PW)], out_hbm.at[1, pl.ds(base, _PW)], sem)
    o1.wait()
    o2.wait()


def kernel(x, cell_id, nodal_values, shape_functions, flag_training, connectivity):
    vals = nodal_values.reshape(-1)
    conn = connectivity.reshape(-1)
    return _interp_sc(cell_id, shape_functions, vals, conn)
